# baseline (speedup 1.0000x reference)
import functools

import jax
import jax.numpy as jnp
from jax import lax
from jax.experimental import pallas as pl
from jax.experimental.pallas import tpu as pltpu

_BT = 128
_N = 64
_C2 = 256
_PAIR = 2 * _N


def _fused_kernel(x_ref, a2_ref, w1_ref, b1_ref, w2_ref, b2_ref, wh_ref,
                  bh_ref, wv2_ref, bv2_ref, wv3_ref, bv3_ref, pm_ref,
                  out_ref, h_scr, p_scr, h1_scr, hw_scr):
    f32 = jnp.float32
    bf16 = jnp.bfloat16
    a2 = a2_ref[...].astype(bf16)
    w1 = w1_ref[...].astype(bf16)
    w2 = w2_ref[...].astype(bf16)
    b1 = b1_ref[...]
    b2 = b2_ref[...]

    _CH = 2048

    def w1_body(i, _):
        r = pl.multiple_of(i * _CH, _CH)
        p_scr[pl.ds(r, _CH), :] = jnp.dot(
            x_ref[pl.ds(r, _CH), :].astype(bf16), w1,
            preferred_element_type=f32).astype(bf16)
        return 0

    def a1_body(i, _):
        r = pl.multiple_of(i * _PAIR, _PAIR)
        h1_scr[pl.ds(r, _PAIR), :] = jnp.maximum(
            jnp.dot(a2, p_scr[pl.ds(r, _PAIR), :],
                    preferred_element_type=f32) + b1, 0.0).astype(bf16)
        return 0

    def w2_body(i, _):
        r = pl.multiple_of(i * _CH, _CH)
        hw_scr[pl.ds(r, _CH), :] = jnp.dot(
            h1_scr[pl.ds(r, _CH), :], w2,
            preferred_element_type=f32).astype(bf16)
        return 0

    def a2_body(i, _):
        parts = []
        for j in range(4):
            r = pl.multiple_of(i * 4 * _PAIR + j * _PAIR, _PAIR)
            h2 = jnp.maximum(
                jnp.dot(a2, hw_scr[pl.ds(r, _PAIR), :],
                        preferred_element_type=f32) + b2, 0.0).astype(bf16)
            parts.append(h2.reshape(2, _N * _C2))
        r2 = pl.multiple_of(i * 8, 8)
        h_scr[pl.ds(r2, 8), :] = jnp.concatenate(parts, axis=0)
        return 0

    n_pairs = _BT // 2
    n_ch = (_BT * _N) // _CH
    with jax.named_scope("pairloop"):
        lax.fori_loop(0, n_ch, w1_body, 0, unroll=n_ch)
        lax.fori_loop(0, n_pairs, a1_body, 0, unroll=64)
        lax.fori_loop(0, n_ch, w2_body, 0, unroll=n_ch)
        lax.fori_loop(0, n_pairs // 4, a2_body, 0, unroll=16)

    hr = h_scr[...]
    with jax.named_scope("head"):
        _KC = 2048
        n_kc = (_N * _C2) // _KC

        acc = jnp.zeros((_BT, 128), f32)
        for k in range(n_kc):
            c = k * _KC
            acc = acc + jnp.dot(hr[:, c:c + _KC],
                                wh_ref[c:c + _KC, :].astype(bf16),
                                preferred_element_type=f32)
        hv = jnp.maximum(acc + bh_ref[...], 0.0)

        v = jnp.maximum(jnp.dot(hv, wv2_ref[...], preferred_element_type=f32)
                        + bv2_ref[...], 0.0)
        v = jnp.sum(v * wv3_ref[...].reshape(1, -1), axis=1, keepdims=True)
        v = v + bv3_ref[...]

        q = v + jnp.dot(hv, pm_ref[...], preferred_element_type=f32)
        out_ref[...] = q[:, :32].astype(out_ref.dtype)


def _full(shape):
    return pl.BlockSpec(shape, lambda i: (0,) * len(shape))


@jax.jit
def _forward(x, a_kron, w1, b1, w2, b2, w_head, b_head, wv2_pad, bv2,
             wv3, bv3, pm):
    B, N, F = x.shape
    a_hat = a_kron[::8, ::8]
    a2 = jnp.zeros((_PAIR, _PAIR), jnp.float32)
    a2 = a2.at[:N, :N].set(a_hat).at[N:, N:].set(a_hat)

    x_rows = x.reshape(B * N, F)
    nb = B // _BT

    out = pl.pallas_call(
        _fused_kernel,
        out_shape=jax.ShapeDtypeStruct((B, 32), jnp.float32),
        grid=(nb,),
        in_specs=[pl.BlockSpec((_BT * N, F), lambda i: (i, 0)),
                  _full(a2.shape), _full(w1.shape), _full(b1.shape),
                  _full(w2.shape), _full(b2.shape), _full(w_head.shape),
                  _full(b_head.shape), _full(wv2_pad.shape), _full(bv2.shape),
                  _full(wv3.shape), _full(bv3.shape), _full(pm.shape)],
        out_specs=pl.BlockSpec((_BT, 32), lambda i: (i, 0)),
        scratch_shapes=[pltpu.VMEM((_BT, _N * _C2), jnp.bfloat16),
                        pltpu.VMEM((_BT * N, 128), jnp.bfloat16),
                        pltpu.VMEM((_BT * N, 128), jnp.bfloat16),
                        pltpu.VMEM((_BT * N, _C2), jnp.bfloat16)],
        compiler_params=pltpu.CompilerParams(
            dimension_semantics=("parallel",)),
    )(x_rows, a2, w1, b1, w2, b2, w_head, b_head, wv2_pad, bv2, wv3, bv3, pm)

    return out.reshape(B, 4, 8)


def kernel(x, a_kron, w1, b1, w2, b2, w_head, b_head, wv2_pad, bv2,
           wv3, bv3, pm):
    return _forward(x, a_kron, w1, b1, w2, b2, w_head, b_head, wv2_pad, bv2,
                    wv3, bv3, pm)


# speedup vs baseline: 1.2053x; 1.2053x over previous
import functools

import jax
import jax.numpy as jnp
from jax import lax
from jax.experimental import pallas as pl
from jax.experimental.pallas import tpu as pltpu

_BT = 128
_N = 64
_C2 = 256
_PAIR = 2 * _N


def _fused_kernel(x_ref, a2_ref, w1_ref, b1_ref, w2_ref, b2_ref, wh_ref,
                  bh_ref, wv2_ref, bv2_ref, wv3_ref, bv3_ref, pm_ref,
                  out_ref, h_scr, p_scr, h1_scr, hw_scr):
    f32 = jnp.float32
    bf16 = jnp.bfloat16
    a2 = a2_ref[...].astype(bf16)
    w1 = w1_ref[...].astype(bf16)
    w2 = w2_ref[...].astype(bf16)
    b1 = b1_ref[...]
    b2 = b2_ref[...]

    _CH = 2048

    def w1_body(i, _):
        r = pl.multiple_of(i * _CH, _CH)
        p_scr[pl.ds(r, _CH), :] = jnp.dot(
            x_ref[pl.ds(r, _CH), :].astype(bf16), w1,
            preferred_element_type=f32).astype(bf16)
        return 0

    def a1_body(i, _):
        r = pl.multiple_of(i * _PAIR, _PAIR)
        h1_scr[pl.ds(r, _PAIR), :] = jnp.maximum(
            jnp.dot(a2, p_scr[pl.ds(r, _PAIR), :],
                    preferred_element_type=f32) + b1, 0.0).astype(bf16)
        return 0

    def w2_body(i, _):
        r = pl.multiple_of(i * _CH, _CH)
        hw_scr[pl.ds(r, _CH), :] = jnp.dot(
            h1_scr[pl.ds(r, _CH), :], w2,
            preferred_element_type=f32).astype(bf16)
        return 0

    def a2_body(i, _):
        r = pl.multiple_of(i * _PAIR, _PAIR)
        h_scr[pl.ds(r, _PAIR), :] = jnp.maximum(
            jnp.dot(a2, hw_scr[pl.ds(r, _PAIR), :],
                    preferred_element_type=f32) + b2, 0.0).astype(bf16)
        return 0

    n_pairs = _BT // 2
    n_ch = (_BT * _N) // _CH
    with jax.named_scope("pairloop"):
        lax.fori_loop(0, n_ch, w1_body, 0, unroll=n_ch)
        lax.fori_loop(0, n_pairs, a1_body, 0, unroll=64)
        lax.fori_loop(0, n_ch, w2_body, 0, unroll=n_ch)
        lax.fori_loop(0, n_pairs, a2_body, 0, unroll=64)

    with jax.named_scope("flatten"):
        hr = h_scr[...].reshape(_BT, _N * _C2)
    with jax.named_scope("head"):
        _KC = 2048
        n_kc = (_N * _C2) // _KC

        acc = jnp.zeros((_BT, 128), f32)
        for k in range(n_kc):
            c = k * _KC
            acc = acc + jnp.dot(hr[:, c:c + _KC],
                                wh_ref[c:c + _KC, :].astype(bf16),
                                preferred_element_type=f32)
        hv = jnp.maximum(acc + bh_ref[...], 0.0)

        v = jnp.maximum(jnp.dot(hv, wv2_ref[...], preferred_element_type=f32)
                        + bv2_ref[...], 0.0)
        v = jnp.sum(v * wv3_ref[...].reshape(1, -1), axis=1, keepdims=True)
        v = v + bv3_ref[...]

        q = v + jnp.dot(hv, pm_ref[...], preferred_element_type=f32)
        out_ref[...] = q[:, :32].astype(out_ref.dtype)


def _full(shape):
    return pl.BlockSpec(shape, lambda i: (0,) * len(shape))


@jax.jit
def _forward(x, a_kron, w1, b1, w2, b2, w_head, b_head, wv2_pad, bv2,
             wv3, bv3, pm):
    B, N, F = x.shape
    a_hat = a_kron[::8, ::8]
    a2 = jnp.zeros((_PAIR, _PAIR), jnp.float32)
    a2 = a2.at[:N, :N].set(a_hat).at[N:, N:].set(a_hat)

    x_rows = x.reshape(B * N, F)
    nb = B // _BT

    out = pl.pallas_call(
        _fused_kernel,
        out_shape=jax.ShapeDtypeStruct((B, 32), jnp.float32),
        grid=(nb,),
        in_specs=[pl.BlockSpec((_BT * N, F), lambda i: (i, 0)),
                  _full(a2.shape), _full(w1.shape), _full(b1.shape),
                  _full(w2.shape), _full(b2.shape), _full(w_head.shape),
                  _full(b_head.shape), _full(wv2_pad.shape), _full(bv2.shape),
                  _full(wv3.shape), _full(bv3.shape), _full(pm.shape)],
        out_specs=pl.BlockSpec((_BT, 32), lambda i: (i, 0)),
        scratch_shapes=[pltpu.VMEM((_BT * N, _C2), jnp.bfloat16),
                        pltpu.VMEM((_BT * N, 128), jnp.bfloat16),
                        pltpu.VMEM((_BT * N, 128), jnp.bfloat16),
                        pltpu.VMEM((_BT * N, _C2), jnp.bfloat16)],
        compiler_params=pltpu.CompilerParams(
            dimension_semantics=("parallel",)),
    )(x_rows, a2, w1, b1, w2, b2, w_head, b_head, wv2_pad, bv2, wv3, bv3, pm)

    return out.reshape(B, 4, 8)


def kernel(x, a_kron, w1, b1, w2, b2, w_head, b_head, wv2_pad, bv2,
           wv3, bv3, pm):
    return _forward(x, a_kron, w1, b1, w2, b2, w_head, b_head, wv2_pad, bv2,
                    wv3, bv3, pm)
